# baseline (speedup 1.0000x reference)
import functools

import jax
import jax.numpy as jnp
from jax import lax
from jax.experimental import pallas as pl
from jax.experimental.pallas import tpu as pltpu
from jax.experimental.pallas import tpu_sc as plsc

NUM_INPUTS = 26
STATE_SIZE = 32
CARDINALITY = 100000
BATCH = 16384

NC = 2
NS = 16
NW = NC * NS
COLS = NUM_INPUTS * STATE_SIZE
CPW = COLS // NW
CHUNK = 4096
NCHUNK = BATCH // CHUNK
NSLOT = 3
L = 16

HALF = 49920
MID = CARDINALITY - 160
TAILPAD = 256
HI = MID - HALF + TAILPAD


def _make_kernel():
    mesh = plsc.VectorSubcoreMesh(core_axis_name="c", subcore_axis_name="s")

    @functools.partial(
        pl.kernel,
        mesh=mesh,
        out_type=jax.ShapeDtypeStruct((NUM_INPUTS, STATE_SIZE, BATCH), jnp.float32),
        scratch_types=[
            pltpu.VMEM((HALF,), jnp.float32),
            pltpu.VMEM((HI,), jnp.float32),
            pltpu.VMEM((BATCH,), jnp.int32),
            pltpu.VMEM((NSLOT * CHUNK,), jnp.float32),
            pltpu.SemaphoreType.DMA,
            pltpu.SemaphoreType.DMA,
            pltpu.SemaphoreType.DMA,
            pltpu.SemaphoreType.DMA,
        ],
        compiler_params=pltpu.CompilerParams(needs_layout_passes=False),
    )
    def col_kernel(
        xt_hbm, tabt_hbm, tail_hbm, out_hbm,
        lo_v, hi_v, idx_v, res_v,
        sem_lo, sem_hi, sem_i, sem_o,
    ):
        wid = lax.axis_index("s") * NC + lax.axis_index("c")

        def fire_lo(t, c):
            pltpu.async_copy(tabt_hbm.at[t, c, pl.ds(0, HALF)], lo_v, sem_lo)

        def drain_lo(t, c):
            pltpu.make_async_copy(
                tabt_hbm.at[t, c, pl.ds(0, HALF)], lo_v, sem_lo
            ).wait()

        def fire_hi(t, c):
            pltpu.async_copy(
                tabt_hbm.at[t, c, pl.ds(HALF, MID - HALF)],
                hi_v.at[pl.ds(0, MID - HALF)],
                sem_hi,
            )
            pltpu.async_copy(
                tail_hbm.at[t * STATE_SIZE + c],
                hi_v.at[pl.ds(MID - HALF, TAILPAD)],
                sem_hi,
            )

        def drain_hi(t, c):
            pltpu.make_async_copy(
                tabt_hbm.at[t, c, pl.ds(HALF, MID - HALF)],
                hi_v.at[pl.ds(0, MID - HALF)],
                sem_hi,
            ).wait()
            pltpu.make_async_copy(
                tail_hbm.at[t * STATE_SIZE + c],
                hi_v.at[pl.ds(MID - HALF, TAILPAD)],
                sem_hi,
            ).wait()

        def lo_gather(j, slot):
            @plsc.parallel_loop(0, CHUNK, step=L, unroll=16)
            def _(i):
                idx = idx_v[pl.ds(j * CHUNK + i, L)]
                lo_idx = jnp.minimum(idx, HALF - 1)
                res_v[pl.ds(slot * CHUNK + i, L)] = plsc.load_gather(lo_v, [lo_idx])

        def hi_merge(j, slot):
            @plsc.parallel_loop(0, CHUNK, step=L, unroll=16)
            def _(i):
                idx = idx_v[pl.ds(j * CHUNK + i, L)]
                hi_idx = jnp.minimum(jnp.maximum(idx - HALF, 0), HI - 1)
                hi_val = plsc.load_gather(hi_v, [hi_idx])
                lo_val = res_v[pl.ds(slot * CHUNK + i, L)]
                res_v[pl.ds(slot * CHUNK + i, L)] = jnp.where(idx >= HALF, hi_val, lo_val)

        def write_res(t, c, j, slot):
            pltpu.async_copy(
                res_v.at[pl.ds(slot * CHUNK, CHUNK)],
                out_hbm.at[t, c, pl.ds(j * CHUNK, CHUNK)],
                sem_o,
            )

        def wait_res(t, c, j, slot):
            pltpu.make_async_copy(
                res_v.at[pl.ds(slot * CHUNK, CHUNK)],
                out_hbm.at[t, c, pl.ds(j * CHUNK, CHUNK)],
                sem_o,
            ).wait()

        def nxt(tau):
            return lax.div(tau + 1, STATE_SIZE), lax.rem(tau + 1, STATE_SIZE)

        def do_col(k, _):
            tau = wid * CPW + k
            t = lax.div(tau, STATE_SIZE)
            c = lax.rem(tau, STATE_SIZE)

            @pl.when(jnp.logical_or(k == 0, c == 0))
            def _():
                pltpu.async_copy(xt_hbm.at[t], idx_v, sem_i)
                pltpu.make_async_copy(xt_hbm.at[t], idx_v, sem_i).wait()

            @pl.when(k == 0)
            def _():
                fire_lo(t, c)
                fire_hi(t, c)

            drain_lo(t, c)

            def lo_chunk(j, _):
                lo_gather(j, j)
                return ()

            lax.fori_loop(0, NSLOT, lo_chunk, (), unroll=False)

            drain_hi(t, c)
            hi_merge(0, 0)
            write_res(t, c, 0, 0)
            wait_res(t, c, 0, 0)
            lo_gather(NCHUNK - 1, 0)

            @pl.when(k + 1 < CPW)
            def _():
                nt, nc_ = nxt(tau)
                fire_lo(nt, nc_)

            hi_merge(1, 1)
            write_res(t, c, 1, 1)
            hi_merge(2, 2)
            write_res(t, c, 2, 2)
            hi_merge(NCHUNK - 1, 0)
            write_res(t, c, NCHUNK - 1, 0)

            @pl.when(k + 1 < CPW)
            def _():
                nt, nc_ = nxt(tau)
                fire_hi(nt, nc_)

            wait_res(t, c, 1, 1)
            wait_res(t, c, 2, 2)
            wait_res(t, c, NCHUNK - 1, 0)
            return ()

        lax.fori_loop(0, CPW, do_col, (), unroll=False)

    return col_kernel


_KERNEL = _make_kernel()


@jax.jit
def kernel(x, tables):
    xt = x.T.astype(jnp.int32)
    tabt = tables.transpose(0, 2, 1)
    tail = jnp.pad(
        lax.slice_in_dim(tabt, MID, CARDINALITY, axis=2),
        ((0, 0), (0, 0), (0, TAILPAD - (CARDINALITY - MID))),
    ).reshape(NUM_INPUTS * STATE_SIZE, TAILPAD)
    out = _KERNEL(xt, tabt, tail)
    return out.transpose(0, 2, 1)
